# baseline (speedup 1.0000x reference)
import functools

import jax
import jax.numpy as jnp
from jax import lax
from jax.experimental import pallas as pl
from jax.experimental.pallas import tpu as pltpu
from jax.experimental.pallas import tpu_sc as plsc

_CHUNK = 128
_L = 16


@functools.cache
def _build_gather(B: int, V: int, D: int):
    info = plsc.get_sparse_core_info()
    NC, NS = info.num_cores, info.num_subcores
    NW = NC * NS
    assert B % (NW * _L) == 0
    b_per_w = B // NW
    e_per_w = b_per_w * D
    n_chunk = e_per_w // _CHUNK
    assert e_per_w % _CHUNK == 0
    mesh = plsc.VectorSubcoreMesh(core_axis_name="c", subcore_axis_name="s")

    @functools.partial(
        pl.kernel,
        mesh=mesh,
        out_type=jax.ShapeDtypeStruct((B * D,), jnp.float32),
        compiler_params=pltpu.CompilerParams(
            use_tc_tiling_on_sc=False, needs_layout_passes=False
        ),
        scratch_types=[
            pltpu.VMEM((b_per_w,), jnp.int32),
            pltpu.VMEM((e_per_w,), jnp.int32),
            pltpu.VMEM((e_per_w,), jnp.float32),
            pltpu.SemaphoreType.DMA,
        ],
    )
    def gather_k(ids_hbm, flat_hbm, out_hbm, idx_v, eidx_v, vals_v, sem):
        wid = lax.axis_index("s") * NC + lax.axis_index("c")
        base = wid * b_per_w
        pltpu.sync_copy(ids_hbm.at[pl.ds(base, b_per_w)], idx_v)
        lanes = lax.iota(jnp.int32, _L)
        for k in range(b_per_w // _L):
            v4 = idx_v[pl.ds(k * _L, _L)] * 4
            pos = lanes * D + (k * _L * D)
            for c in range(D):
                plsc.store_scatter(eidx_v, [pos + c], v4 + c)
        copies = [
            pltpu.async_copy(
                flat_hbm.at[eidx_v.at[pl.ds(j * _CHUNK, _CHUNK)]],
                vals_v.at[pl.ds(j * _CHUNK, _CHUNK)],
                sem,
            )
            for j in range(n_chunk)
        ]
        for c in copies:
            c.wait()
        pltpu.sync_copy(vals_v, out_hbm.at[pl.ds(base * D, e_per_w)])

    return gather_k


def kernel(ids, data):
    B, = ids.shape
    V, D = data.shape
    gather_k = _build_gather(B, V, D)
    out_flat = gather_k(ids.astype(jnp.int32), data.reshape(V * D))
    return out_flat.reshape(B, D)


# speedup vs baseline: 1.0413x; 1.0413x over previous
import functools

import jax
import jax.numpy as jnp
from jax import lax
from jax.experimental import pallas as pl
from jax.experimental.pallas import tpu as pltpu
from jax.experimental.pallas import tpu_sc as plsc

_CHUNK = 128
_L = 16


@functools.cache
def _build_gather(B: int, V: int, D: int):
    info = plsc.get_sparse_core_info()
    NC, NS = info.num_cores, info.num_subcores
    NW = NC * NS
    assert B % (NW * _L) == 0 and V % 2 == 0
    assert D == 4
    b_per_w = B // NW
    e_per_w = b_per_w * D
    n_chunk = b_per_w // _CHUNK
    assert b_per_w % _CHUNK == 0
    mesh = plsc.VectorSubcoreMesh(core_axis_name="c", subcore_axis_name="s")

    @functools.partial(
        pl.kernel,
        mesh=mesh,
        out_type=jax.ShapeDtypeStruct((B, D), jnp.float32),
        compiler_params=pltpu.CompilerParams(
            use_tc_tiling_on_sc=False, needs_layout_passes=False
        ),
        scratch_types=[
            pltpu.VMEM((b_per_w,), jnp.int32),
            pltpu.VMEM((b_per_w,), jnp.int32),
            pltpu.VMEM((b_per_w, 2 * D), jnp.float32),
            pltpu.VMEM((b_per_w, D), jnp.float32),
            pltpu.SemaphoreType.DMA,
        ],
    )
    def gather_k(ids_hbm, pairs_hbm, out_hbm, idx_v, idx8_v, rows8_v,
                 vals_v, sem):
        wid = lax.axis_index("s") * NC + lax.axis_index("c")
        base = wid * b_per_w
        lanes = lax.iota(jnp.int32, _L)

        pltpu.sync_copy(ids_hbm.at[pl.ds(base, b_per_w)], idx_v)
        for k in range(b_per_w // _L):
            idx8_v[pl.ds(k * _L, _L)] = idx_v[pl.ds(k * _L, _L)] >> 1

        copies = [
            pltpu.async_copy(
                pairs_hbm.at[idx8_v.at[pl.ds(j * _CHUNK, _CHUNK)]],
                rows8_v.at[pl.ds(j * _CHUNK, _CHUNK)],
                sem,
            )
            for j in range(n_chunk)
        ]
        for cpy in copies:
            cpy.wait()

        def compact(k, carry):
            e = lanes + k * _L
            rowv = e >> 2
            idsv = plsc.load_gather(idx_v, [rowv])
            colv = ((idsv & 1) << 2) + (e & 3)
            v = plsc.load_gather(rows8_v, [rowv, colv])
            plsc.store_scatter(vals_v, [rowv, e & 3], v)
            return carry

        lax.fori_loop(0, e_per_w // _L, compact, 0, unroll=4)
        pltpu.sync_copy(vals_v, out_hbm.at[pl.ds(base, b_per_w)])

    return gather_k


def kernel(ids, data):
    B, = ids.shape
    V, D = data.shape
    gather_k = _build_gather(B, V, D)
    pairs = data.reshape(V // 2, 2 * D)
    return gather_k(ids.astype(jnp.int32), pairs)
